# baseline (speedup 1.0000x reference)
import functools

import jax
import jax.numpy as jnp
from jax import lax
from jax.experimental import pallas as pl
from jax.experimental.pallas import tpu as pltpu
from jax.experimental.pallas import tpu_sc as plsc

VOCAB = 1000
DP = 1024
B = 1024
S = 50
N = B * S
NC = 2
NS = 16
NW = NC * NS
PER_W = N // NW
K = 16
NG = PER_W // K
K2 = 32
NG2 = PER_W // K2
LANES = 16

_MESH = dict(core_axis_name="c", subcore_axis_name="s")


def _sc_gather_padded(idx, table_pad):
  @functools.partial(
      pl.kernel,
      mesh=plsc.VectorSubcoreMesh(**_MESH),
      out_type=jax.ShapeDtypeStruct((N // K, K, DP), jnp.float32),
      scratch_types=[
          pltpu.VMEM((NG, K), jnp.int32),
          pltpu.VMEM((K, DP), jnp.float32),
          pltpu.VMEM((K, DP), jnp.float32),
          pltpu.SemaphoreType.DMA,
          pltpu.SemaphoreType.DMA,
          pltpu.SemaphoreType.DMA,
          pltpu.SemaphoreType.DMA,
      ],
  )
  def k(idx_hbm, table_hbm, pad_hbm, idx_v, pad0, pad1, gsem0, gsem1,
        ssem0, ssem1):
    wid = lax.axis_index("s") * NC + lax.axis_index("c")
    gbase = wid * NG
    pltpu.sync_copy(idx_hbm.at[wid], idx_v)

    pads = (pad0, pad1)
    gsems = (gsem0, gsem1)
    ssems = (ssem0, ssem1)

    def fire_gather(g, p):
      pltpu.async_copy(table_hbm.at[idx_v.at[g]], pads[p], gsems[p])

    def out_slice(g):
      return pad_hbm.at[gbase + g]

    fire_gather(0, 0)
    fire_gather(1, 1)

    @pl.loop(0, NG, step=2)
    def body(gg):
      for p in range(2):
        g = gg + p
        pltpu.make_async_copy(table_hbm.at[idx_v.at[g]], pads[p],
                              gsems[p]).wait()
        pltpu.async_copy(pads[p], out_slice(g), ssems[p])

        @pl.when(g + 2 < NG)
        def _():
          pltpu.make_async_copy(pads[p], out_slice(g), ssems[p]).wait()
          fire_gather(g + 2, p)

    for p in range(2):
      pltpu.make_async_copy(pads[p], out_slice(NG - 2 + p), ssems[p]).wait()

  return k(idx, table_pad)


def _sc_depad(pad_flat):
  @functools.partial(
      pl.kernel,
      mesh=plsc.VectorSubcoreMesh(**_MESH),
      out_type=jax.ShapeDtypeStruct((N * VOCAB,), jnp.float32),
      scratch_types=[
          pltpu.VMEM((K2 * DP,), jnp.float32),
          pltpu.VMEM((K2 * DP,), jnp.float32),
          pltpu.VMEM((K2 * VOCAB,), jnp.float32),
          pltpu.VMEM((K2 * VOCAB,), jnp.float32),
          pltpu.SemaphoreType.DMA,
          pltpu.SemaphoreType.DMA,
          pltpu.SemaphoreType.DMA,
          pltpu.SemaphoreType.DMA,
      ],
  )
  def k(pad_hbm, out_hbm, in0, in1, cmp0, cmp1, isem0, isem1, osem0, osem1):
    wid = lax.axis_index("s") * NC + lax.axis_index("c")
    row0 = wid * PER_W

    ins = (in0, in1)
    cmps = (cmp0, cmp1)
    isems = (isem0, isem1)
    osems = (osem0, osem1)

    def in_slice(g):
      off = pl.multiple_of((row0 + g * K2) * DP, 8)
      return pad_hbm.at[pl.ds(off, K2 * DP)]

    def out_slice(g):
      off = pl.multiple_of((row0 + g * K2) * VOCAB, 8)
      return out_hbm.at[pl.ds(off, K2 * VOCAB)]

    def fire_in(g, p):
      pltpu.async_copy(in_slice(g), ins[p], isems[p])

    fire_in(0, 0)
    fire_in(1, 1)

    @pl.loop(0, NG2, step=2)
    def body(gg):
      for p in range(2):
        g = gg + p
        pltpu.make_async_copy(in_slice(g), ins[p], isems[p]).wait()

        @pl.when(g >= 2)
        def _():
          pltpu.make_async_copy(cmps[p], out_slice(g - 2), osems[p]).wait()

        @pl.loop(0, K2)
        def depad(r):
          src = pl.multiple_of(r * DP, 8)
          dst = pl.multiple_of(r * VOCAB, 8)
          for c in range(VOCAB // LANES):
            cmps[p][pl.ds(dst + c * LANES, LANES)] = (
                ins[p][pl.ds(src + c * LANES, LANES)])
          cmps[p][pl.ds(dst + VOCAB - LANES, LANES)] = (
              ins[p][pl.ds(src + VOCAB - LANES, LANES)])

        pltpu.async_copy(cmps[p], out_slice(g), osems[p])

        @pl.when(g + 2 < NG2)
        def _():
          fire_in(g + 2, p)

    for p in range(2):
      pltpu.make_async_copy(cmps[p], out_slice(NG2 - 2 + p), osems[p]).wait()

  return k(pad_flat)


def kernel(x, tkn_emb_table):
  idx = x.reshape(NW, NG, K).astype(jnp.int32)
  table_p = jnp.pad(tkn_emb_table, ((0, 0), (0, DP - VOCAB)))
  padded = _sc_gather_padded(idx, table_p)
  out = _sc_depad(padded.reshape(N * DP))
  return out.reshape(B, S, VOCAB)


# speedup vs baseline: 1.7647x; 1.7647x over previous
import functools

import jax
import jax.numpy as jnp
from jax import lax
from jax.experimental import pallas as pl
from jax.experimental.pallas import tpu as pltpu
from jax.experimental.pallas import tpu_sc as plsc

VOCAB = 1000
B = 1024
S = 50
N = B * S
NC = 2
NS = 16
NW = NC * NS
PER_W = N // NW
K = 32
NG = PER_W // K


def _sc_gather(idx, table_flat):
  mesh = plsc.VectorSubcoreMesh(core_axis_name="c", subcore_axis_name="s")

  @functools.partial(
      pl.kernel,
      mesh=mesh,
      out_type=jax.ShapeDtypeStruct((N * VOCAB,), jnp.float32),
      scratch_types=[
          pltpu.VMEM((PER_W,), jnp.int32),
          pltpu.VMEM((K * VOCAB,), jnp.float32),
          pltpu.VMEM((K * VOCAB,), jnp.float32),
          pltpu.SemaphoreType.DMA,
          pltpu.SemaphoreType.DMA,
      ],
  )
  def k(idx_hbm, table_hbm, out_hbm, idx_v, buf0, buf1, gsem0, gsem1):
    wid = lax.axis_index("s") * NC + lax.axis_index("c")
    base = wid * PER_W
    pltpu.sync_copy(idx_hbm.at[pl.ds(base, PER_W)], idx_v)

    bufs = (buf0, buf1)
    gsems = (gsem0, gsem1)

    def fire_gathers(g, p):
      for h in range(K // 16):
        iv = idx_v[pl.ds(pl.multiple_of(g * K + h * 16, 16), 16)] * VOCAB
        for t in range(16):
          src = pl.multiple_of(iv[t], 8)
          pltpu.async_copy(table_hbm.at[pl.ds(src, VOCAB)],
                           bufs[p].at[pl.ds((h * 16 + t) * VOCAB, VOCAB)],
                           gsems[p])

    def out_slice(g):
      off = pl.multiple_of((base + g * K) * VOCAB, 8)
      return out_hbm.at[pl.ds(off, K * VOCAB)]

    fire_gathers(0, 0)
    fire_gathers(1, 1)

    @pl.loop(0, NG, step=2)
    def body(gg):
      for p in range(2):
        g = gg + p
        pltpu.make_async_copy(out_slice(g), bufs[p], gsems[p]).wait()

        pltpu.sync_copy(bufs[p], out_slice(g))

        @pl.when(g + 2 < NG)
        def _():
          fire_gathers(g + 2, p)

  return k(idx, table_flat)


def kernel(x, tkn_emb_table):
  idx = x.reshape(-1).astype(jnp.int32)
  out = _sc_gather(idx, tkn_emb_table.reshape(-1))
  return out.reshape(B, S, VOCAB)
